# baseline (speedup 1.0000x reference)
import functools

import jax
import jax.numpy as jnp
from jax import lax
from jax.experimental import pallas as pl
from jax.experimental.pallas import tpu as pltpu
from jax.experimental.pallas import tpu_sc as plsc

D_MODEL = 64
D_PAD = 128
SEQ = 200
BATCH = 4096
TOK_BLK = 256
BLKS = BATCH // TOK_BLK
ITEMS = SEQ * BLKS
NUM_CORES = 2
NUM_SUBCORES = 16
NUM_WORKERS = NUM_CORES * NUM_SUBCORES
ITEMS_PER_W = ITEMS // NUM_WORKERS


VOCAB = 1000000
VTILES = VOCAB // 128
VTAIL = VOCAB - VTILES * 128


def _transpose_table(wt, wtail):
    mesh = plsc.VectorSubcoreMesh(core_axis_name="c", subcore_axis_name="s")

    @functools.partial(
        pl.kernel,
        mesh=mesh,
        out_type=jax.ShapeDtypeStruct((VOCAB, D_PAD), jnp.float32),
        scratch_types=[
            pltpu.VMEM((2, D_MODEL, 128), jnp.float32),
            pltpu.VMEM((2, 128, D_PAD), jnp.float32),
            [pltpu.SemaphoreType.DMA] * 2,
        ],
        compiler_params=pltpu.CompilerParams(use_tc_tiling_on_sc=True,
                                             needs_layout_passes=False),
    )
    def k(wt_hbm, wtail_hbm, wp_hbm, a_v, b_v, sems):
        wid = lax.axis_index("s") * NUM_CORES + lax.axis_index("c")
        lane = lax.iota(jnp.int32, 16)
        rot = [lax.rem(lane + r, 16) for r in range(16)]
        n_i = jnp.where(wid < VTILES % NUM_WORKERS,
                        VTILES // NUM_WORKERS + 1,
                        VTILES // NUM_WORKERS)

        def tc_of(i):
            return wid + i * NUM_WORKERS

        def fetch(i, b):
            pltpu.async_copy(wt_hbm.at[:, pl.ds(tc_of(i) * 128, 128)],
                             a_v.at[b], sems[b])

        def wait_fetch(i, b):
            pltpu.make_async_copy(
                wt_hbm.at[:, pl.ds(tc_of(i) * 128, 128)],
                a_v.at[b], sems[b]).wait()

        def flush(i, b):
            src = a_v.at[b]
            dst = b_v.at[b]

            @plsc.parallel_loop(0, 8, unroll=2)
            def trans(vg):
                v0 = vg * 16
                for dg in range(D_MODEL // 16):
                    d0 = dg * 16
                    for r in range(16):
                        dd = rot[r] + d0
                        vals = plsc.load_gather(src, [dd, v0 + lane])
                        plsc.store_scatter(dst, [v0 + lane, dd], vals)

            pltpu.sync_copy(dst, wp_hbm.at[pl.ds(tc_of(i) * 128, 128), :])

        @pl.when(n_i > 0)
        def _main():
            fetch(0, 0)

            def body2(g, carry):
                i = g * 2
                wait_fetch(i, 0)
                fetch(i + 1, 1)
                flush(i, 0)
                wait_fetch(i + 1, 1)
                fetch(i + 2, 0)
                flush(i + 1, 1)
                return carry

            n_pairs = (n_i - 2) // 2
            lax.fori_loop(0, n_pairs, body2, 0)

            def tail(i, carry):
                wait_fetch(i, 0)
                flush(i, 0)
                nxt = i + 1

                @pl.when(nxt < n_i)
                def _():
                    fetch(nxt, 0)
                return carry

            lax.fori_loop(n_pairs * 2, n_i, tail, 0)

        @pl.when(wid == 0)
        def _tail():
            pltpu.sync_copy(wtail_hbm, b_v.at[0, pl.ds(0, VTAIL), :])
            pltpu.sync_copy(b_v.at[0, pl.ds(0, VTAIL), :],
                            wp_hbm.at[pl.ds(VTILES * 128, VTAIL), :])

    return k(wt, wtail)


def _gather_embed(wp, ids_t):
    mesh = plsc.VectorSubcoreMesh(core_axis_name="c", subcore_axis_name="s")

    @functools.partial(
        pl.kernel,
        mesh=mesh,
        out_type=jax.ShapeDtypeStruct((SEQ, D_MODEL, BATCH), jnp.float32),
        scratch_types=[
            pltpu.VMEM((2, 2, 128), jnp.int32),
            pltpu.VMEM((2, TOK_BLK, D_PAD), jnp.float32),
            pltpu.VMEM((D_MODEL, TOK_BLK), jnp.float32),
            [pltpu.SemaphoreType.DMA] * 2,
        ],
        compiler_params=pltpu.CompilerParams(use_tc_tiling_on_sc=True,
                                             needs_layout_passes=False),
    )
    def k(table_hbm, idx_hbm, out_hbm, idx_v, g_v, t_v, sems):
        wid = lax.axis_index("s") * NUM_CORES + lax.axis_index("c")
        item0 = wid * ITEMS_PER_W
        lane = lax.iota(jnp.int32, 16)
        rot = [lax.rem(lane + r, 16) for r in range(16)]

        def stage(i, b):
            item = item0 + i
            s = item // BLKS
            b0 = (item % BLKS) * TOK_BLK
            with jax.named_scope("idx_copy"):
                for j in range(2):
                    pltpu.sync_copy(idx_hbm.at[s, pl.ds(b0 + j * 128, 128)],
                                    idx_v.at[b, j])
            with jax.named_scope("fire_gather"):
                for j in range(2):
                    pltpu.async_copy(table_hbm.at[idx_v.at[b, j]],
                                     g_v.at[b, pl.ds(j * 128, 128)],
                                     sems[b])

        def drain(b):
            with jax.named_scope("drain_gather"):
                for j in range(2):
                    pltpu.make_async_copy(
                        table_hbm.at[idx_v.at[b, j]],
                        g_v.at[b, pl.ds(j * 128, 128)],
                        sems[b],
                    ).wait()

        def flush(i, b):
            item = item0 + i
            s = item // BLKS
            b0 = (item % BLKS) * TOK_BLK
            src = g_v.at[b]

            with jax.named_scope("transpose"):
                @plsc.parallel_loop(0, TOK_BLK // 16, unroll=2)
                def trans(tg):
                    row = lane + tg * 16
                    for dg in range(D_MODEL // 16):
                        d0 = dg * 16
                        for r in range(16):
                            col = rot[r] + d0
                            vals = plsc.load_gather(src, [row, col])
                            plsc.store_scatter(t_v, [col, row], vals)
            with jax.named_scope("store_out"):
                pltpu.sync_copy(t_v, out_hbm.at[s, :, pl.ds(b0, TOK_BLK)])

        stage(0, 0)

        def body2(g, carry):
            i = g * 2
            drain(0)
            stage(i + 1, 1)
            flush(i, 0)
            drain(1)
            stage(i + 2, 0)
            flush(i + 1, 1)
            return carry

        lax.fori_loop(0, (ITEMS_PER_W - 2) // 2, body2, 0)
        i_last = ITEMS_PER_W - 2
        drain(0)
        stage(i_last + 1, 1)
        flush(i_last, 0)
        drain(1)
        flush(i_last + 1, 1)

    return k(wp, ids_t)


def kernel(input_ids, weight):
    wtail = jnp.pad(weight[VTILES * 128:, :], ((0, 0), (0, D_PAD - D_MODEL)))
    wp = _transpose_table(weight.T, wtail)
    ids_t = input_ids.T.astype(jnp.int32)
    out = _gather_embed(wp, ids_t)
    return out.transpose(2, 0, 1)


# speedup vs baseline: 1.1723x; 1.1723x over previous
import functools

import jax
import jax.numpy as jnp
from jax import lax
from jax.experimental import pallas as pl
from jax.experimental.pallas import tpu as pltpu
from jax.experimental.pallas import tpu_sc as plsc

D_MODEL = 64
D_PAD = 128
SEQ = 200
BATCH = 4096
TOK_BLK = 256
BLKS = BATCH // TOK_BLK
ITEMS = SEQ * BLKS
NUM_CORES = 2
NUM_SUBCORES = 16
NUM_WORKERS = NUM_CORES * NUM_SUBCORES
ITEMS_PER_W = ITEMS // NUM_WORKERS

VOCAB = 1000000
VTILES = VOCAB // 128
VTAIL = VOCAB - VTILES * 128


def _diag_transpose(src, dst, lane, rot, n_row_groups, n_col_groups):
    @plsc.parallel_loop(0, n_row_groups, unroll=2)
    def trans(rg):
        r0 = rg * 16
        row = lane + r0
        for cg in range(n_col_groups):
            c0 = cg * 16
            for r in range(16):
                col = rot[r] + c0
                vals = plsc.load_gather(src, [row, col])
                plsc.store_scatter(dst, [col, row], vals)


def _transpose_table(wt, wtail):
    mesh = plsc.VectorSubcoreMesh(core_axis_name="c", subcore_axis_name="s")

    @functools.partial(
        pl.kernel,
        mesh=mesh,
        out_type=jax.ShapeDtypeStruct((VOCAB, D_PAD), jnp.float32),
        scratch_types=[
            pltpu.VMEM((2, D_MODEL, 128), jnp.float32),
            pltpu.VMEM((2, 128, D_PAD), jnp.float32),
            [pltpu.SemaphoreType.DMA] * 4,
        ],
        compiler_params=pltpu.CompilerParams(use_tc_tiling_on_sc=True,
                                             needs_layout_passes=False),
    )
    def k(wt_hbm, wtail_hbm, wp_hbm, a_v, b_v, sems):
        sems_f = [sems[0], sems[1]]
        sems_s = [sems[2], sems[3]]
        wid = lax.axis_index("s") * NUM_CORES + lax.axis_index("c")
        lane = lax.iota(jnp.int32, 16)
        rot = [lax.rem(lane + r, 16) for r in range(16)]
        n_i = jnp.where(wid < VTILES % NUM_WORKERS,
                        VTILES // NUM_WORKERS + 1,
                        VTILES // NUM_WORKERS)

        def tc_of(i):
            return wid + i * NUM_WORKERS

        def fetch(i, b):
            pltpu.async_copy(wt_hbm.at[:, pl.ds(tc_of(i) * 128, 128)],
                             a_v.at[b], sems_f[b])

        def wait_fetch(i, b):
            pltpu.make_async_copy(
                wt_hbm.at[:, pl.ds(tc_of(i) * 128, 128)],
                a_v.at[b], sems_f[b]).wait()

        def store(i, b):
            pltpu.async_copy(b_v.at[b],
                             wp_hbm.at[pl.ds(tc_of(i) * 128, 128), :],
                             sems_s[b])

        def wait_store(b):
            pltpu.make_async_copy(b_v.at[b],
                                  wp_hbm.at[pl.ds(0, 128), :],
                                  sems_s[b]).wait()

        def flush(i, b):
            wait_store(b)
            _diag_transpose(a_v.at[b], b_v.at[b], lane, rot, 8,
                            D_MODEL // 16)
            store(i, b)

        @pl.when(n_i > 0)
        def _main():
            fetch(0, 0)
            store(0, 0)
            store(1, 1)

            def body2(g, carry):
                i = g * 2
                wait_fetch(i, 0)
                fetch(i + 1, 1)
                flush(i, 0)
                wait_fetch(i + 1, 1)
                fetch(i + 2, 0)
                flush(i + 1, 1)
                return carry

            n_pairs = (n_i - 2) // 2
            lax.fori_loop(0, n_pairs, body2, 0)
            wait_store(0)
            wait_store(1)

            def tail(i, carry):
                wait_fetch(i, 0)
                _diag_transpose(a_v.at[0], b_v.at[0], lane, rot, 8,
                                D_MODEL // 16)
                pltpu.sync_copy(b_v.at[0],
                                wp_hbm.at[pl.ds(tc_of(i) * 128, 128), :])
                nxt = i + 1

                @pl.when(nxt < n_i)
                def _():
                    fetch(nxt, 0)
                return carry

            lax.fori_loop(n_pairs * 2, n_i, tail, 0)

        @pl.when(wid == 0)
        def _tail():
            pltpu.sync_copy(wtail_hbm, b_v.at[0, pl.ds(0, VTAIL), :])
            pltpu.sync_copy(b_v.at[0, pl.ds(0, VTAIL), :],
                            wp_hbm.at[pl.ds(VTILES * 128, VTAIL), :])

    return k(wt, wtail)


def _gather_embed(wp, ids_t):
    mesh = plsc.VectorSubcoreMesh(core_axis_name="c", subcore_axis_name="s")

    @functools.partial(
        pl.kernel,
        mesh=mesh,
        out_type=jax.ShapeDtypeStruct((SEQ, D_MODEL, BATCH), jnp.float32),
        scratch_types=[
            pltpu.VMEM((2, 2, 128), jnp.int32),
            pltpu.VMEM((2, TOK_BLK, D_PAD), jnp.float32),
            pltpu.VMEM((2, D_MODEL, TOK_BLK), jnp.float32),
            [pltpu.SemaphoreType.DMA] * 4,
        ],
        compiler_params=pltpu.CompilerParams(use_tc_tiling_on_sc=True,
                                             needs_layout_passes=False),
    )
    def k(table_hbm, idx_hbm, out_hbm, idx_v, g_v, t_v, sems):
        sems_g = [sems[0], sems[1]]
        sems_s = [sems[2], sems[3]]
        wid = lax.axis_index("s") * NUM_CORES + lax.axis_index("c")
        item0 = wid * ITEMS_PER_W
        lane = lax.iota(jnp.int32, 16)
        rot = [lax.rem(lane + r, 16) for r in range(16)]

        def pos(i):
            item = item0 + i
            return item // BLKS, (item % BLKS) * TOK_BLK

        def stage(i, b):
            s, b0 = pos(i)
            for j in range(2):
                pltpu.sync_copy(idx_hbm.at[s, pl.ds(b0 + j * 128, 128)],
                                idx_v.at[b, j])
            for j in range(2):
                pltpu.async_copy(table_hbm.at[idx_v.at[b, j]],
                                 g_v.at[b, pl.ds(j * 128, 128)],
                                 sems_g[b])

        def drain(b):
            for j in range(2):
                pltpu.make_async_copy(
                    table_hbm.at[idx_v.at[b, j]],
                    g_v.at[b, pl.ds(j * 128, 128)],
                    sems_g[b]).wait()

        def store(i, b):
            s, b0 = pos(i)
            pltpu.async_copy(t_v.at[b],
                             out_hbm.at[s, :, pl.ds(b0, TOK_BLK)],
                             sems_s[b])

        def wait_store(b):
            pltpu.make_async_copy(t_v.at[b],
                                  out_hbm.at[0, :, pl.ds(0, TOK_BLK)],
                                  sems_s[b]).wait()

        def flush(i, b):
            wait_store(b)
            _diag_transpose(g_v.at[b], t_v.at[b], lane, rot,
                            TOK_BLK // 16, D_MODEL // 16)
            store(i, b)

        stage(0, 0)
        store(0, 0)
        store(1, 1)

        def body2(g, carry):
            i = g * 2
            drain(0)
            stage(i + 1, 1)
            flush(i, 0)
            drain(1)
            stage(i + 2, 0)
            flush(i + 1, 1)
            return carry

        lax.fori_loop(0, (ITEMS_PER_W - 2) // 2, body2, 0)
        i_last = ITEMS_PER_W - 2
        drain(0)
        stage(i_last + 1, 1)
        flush(i_last, 0)
        drain(1)
        flush(i_last + 1, 1)
        wait_store(0)
        wait_store(1)

    return k(wp, ids_t)


def kernel(input_ids, weight):
    wtail = jnp.pad(weight[VTILES * 128:, :], ((0, 0), (0, D_PAD - D_MODEL)))
    wp = _transpose_table(weight.T, wtail)
    ids_t = input_ids.T.astype(jnp.int32)
    out = _gather_embed(wp, ids_t)
    return out.transpose(2, 0, 1)


# speedup vs baseline: 1.2664x; 1.0802x over previous
import functools

import jax
import jax.numpy as jnp
from jax import lax
from jax.experimental import pallas as pl
from jax.experimental.pallas import tpu as pltpu
from jax.experimental.pallas import tpu_sc as plsc

D_MODEL = 64
D_PAD = 128
SEQ = 200
BATCH = 4096
TOK_BLK = 256
BLKS = BATCH // TOK_BLK
ITEMS = SEQ * BLKS
NUM_CORES = 2
NUM_SUBCORES = 16
NUM_WORKERS = NUM_CORES * NUM_SUBCORES
ITEMS_PER_W = ITEMS // NUM_WORKERS

VOCAB = 1000000
VTILES = VOCAB // 128
VTAIL = VOCAB - VTILES * 128


def _diag_transpose(src, dst, lane, rot, n_row_groups, n_col_groups):
    @plsc.parallel_loop(0, n_row_groups, unroll=2)
    def trans(rg):
        r0 = rg * 16
        row = lane + r0
        for cg in range(n_col_groups):
            c0 = cg * 16
            for r in range(16):
                col = rot[r] + c0
                vals = plsc.load_gather(src, [row, col])
                plsc.store_scatter(dst, [col, row], vals)


def _transpose_table(wt, wtail):
    mesh = plsc.VectorSubcoreMesh(core_axis_name="c", subcore_axis_name="s")

    @functools.partial(
        pl.kernel,
        mesh=mesh,
        out_type=jax.ShapeDtypeStruct((VOCAB, D_PAD), jnp.float32),
        scratch_types=[
            pltpu.VMEM((2, D_MODEL, 128), jnp.float32),
            pltpu.VMEM((2, 128, D_PAD), jnp.float32),
            [pltpu.SemaphoreType.DMA] * 4,
        ],
        compiler_params=pltpu.CompilerParams(use_tc_tiling_on_sc=True,
                                             needs_layout_passes=False),
    )
    def k(wt_hbm, wtail_hbm, wp_hbm, a_v, b_v, sems):
        sems_f = [sems[0], sems[1]]
        sems_s = [sems[2], sems[3]]
        wid = lax.axis_index("s") * NUM_CORES + lax.axis_index("c")
        lane = lax.iota(jnp.int32, 16)
        rot = [lax.rem(lane + r, 16) for r in range(16)]
        n_i = jnp.where(wid < VTILES % NUM_WORKERS,
                        VTILES // NUM_WORKERS + 1,
                        VTILES // NUM_WORKERS)

        def tc_of(i):
            return wid + i * NUM_WORKERS

        def fetch(i, b):
            pltpu.async_copy(wt_hbm.at[:, pl.ds(tc_of(i) * 128, 128)],
                             a_v.at[b], sems_f[b])

        def wait_fetch(i, b):
            pltpu.make_async_copy(
                wt_hbm.at[:, pl.ds(tc_of(i) * 128, 128)],
                a_v.at[b], sems_f[b]).wait()

        def store(i, b):
            pltpu.async_copy(b_v.at[b],
                             wp_hbm.at[pl.ds(tc_of(i) * 128, 128), :],
                             sems_s[b])

        def wait_store(b):
            pltpu.make_async_copy(b_v.at[b],
                                  wp_hbm.at[pl.ds(0, 128), :],
                                  sems_s[b]).wait()

        def flush(i, b):
            wait_store(b)
            _diag_transpose(a_v.at[b], b_v.at[b], lane, rot, 8,
                            D_MODEL // 16)
            store(i, b)

        @pl.when(n_i > 0)
        def _main():
            fetch(0, 0)
            store(0, 0)
            store(1, 1)

            def body2(g, carry):
                i = g * 2
                wait_fetch(i, 0)
                fetch(i + 1, 1)
                flush(i, 0)
                wait_fetch(i + 1, 1)
                fetch(i + 2, 0)
                flush(i + 1, 1)
                return carry

            n_pairs = (n_i - 2) // 2
            lax.fori_loop(0, n_pairs, body2, 0)
            wait_store(0)
            wait_store(1)

            def tail(i, carry):
                wait_fetch(i, 0)
                _diag_transpose(a_v.at[0], b_v.at[0], lane, rot, 8,
                                D_MODEL // 16)
                pltpu.sync_copy(b_v.at[0],
                                wp_hbm.at[pl.ds(tc_of(i) * 128, 128), :])
                nxt = i + 1

                @pl.when(nxt < n_i)
                def _():
                    fetch(nxt, 0)
                return carry

            lax.fori_loop(n_pairs * 2, n_i, tail, 0)

        @pl.when(wid == 0)
        def _tail():
            pltpu.sync_copy(wtail_hbm, b_v.at[0, pl.ds(0, VTAIL), :])
            pltpu.sync_copy(b_v.at[0, pl.ds(0, VTAIL), :],
                            wp_hbm.at[pl.ds(VTILES * 128, VTAIL), :])

    return k(wt, wtail)


def _gather_embed(wp, ids_t):
    mesh = plsc.VectorSubcoreMesh(core_axis_name="c", subcore_axis_name="s")

    @functools.partial(
        pl.kernel,
        mesh=mesh,
        out_type=jax.ShapeDtypeStruct((SEQ, D_MODEL, BATCH), jnp.float32),
        scratch_types=[
            pltpu.VMEM((2, 2, 128), jnp.int32),
            pltpu.VMEM((2, TOK_BLK, D_PAD), jnp.float32),
            pltpu.VMEM((2, D_MODEL, TOK_BLK), jnp.float32),
            [pltpu.SemaphoreType.DMA] * 4,
        ],
        compiler_params=pltpu.CompilerParams(use_tc_tiling_on_sc=True,
                                             needs_layout_passes=False),
    )
    def k(table_hbm, idx_hbm, out_hbm, idx_v, g_v, t_v, sems):
        sems_g = [sems[0], sems[1]]
        sems_s = [sems[2], sems[3]]
        wid = lax.axis_index("s") * NUM_CORES + lax.axis_index("c")
        item0 = wid * ITEMS_PER_W
        lane = lax.iota(jnp.int32, 16)
        rot = [lax.rem(lane + r, 16) for r in range(16)]

        def pos(i):
            item = item0 + i
            return item // BLKS, (item % BLKS) * TOK_BLK

        def stage(i, b):
            s, b0 = pos(i)
            for j in range(2):
                pltpu.sync_copy(idx_hbm.at[s, pl.ds(b0 + j * 128, 128)],
                                idx_v.at[b, j])
            for j in range(2):
                pltpu.async_copy(table_hbm.at[idx_v.at[b, j]],
                                 g_v.at[b, pl.ds(j * 128, 128)],
                                 sems_g[b])

        def drain(b):
            for j in range(2):
                pltpu.make_async_copy(
                    table_hbm.at[idx_v.at[b, j]],
                    g_v.at[b, pl.ds(j * 128, 128)],
                    sems_g[b]).wait()

        def store(i, b):
            s, b0 = pos(i)
            pltpu.async_copy(t_v.at[b],
                             out_hbm.at[s, :, pl.ds(b0, TOK_BLK)],
                             sems_s[b])

        def wait_store(b):
            pltpu.make_async_copy(t_v.at[b],
                                  out_hbm.at[0, :, pl.ds(0, TOK_BLK)],
                                  sems_s[b]).wait()

        def flush(i, b):
            wait_store(b)
            _diag_transpose(g_v.at[b], t_v.at[b], lane, rot,
                            TOK_BLK // 16, D_MODEL // 16)
            store(i, b)

        stage(0, 0)
        store(0, 0)
        store(1, 1)

        def body2(g, carry):
            i = g * 2
            drain(0)
            stage(i + 1, 1)
            flush(i, 0)
            drain(1)
            stage(i + 2, 0)
            flush(i + 1, 1)
            return carry

        lax.fori_loop(0, (ITEMS_PER_W - 2) // 2, body2, 0)
        i_last = ITEMS_PER_W - 2
        drain(0)
        stage(i_last + 1, 1)
        flush(i_last, 0)
        drain(1)
        flush(i_last + 1, 1)
        wait_store(0)
        wait_store(1)

    return k(wp, ids_t)


def kernel(input_ids, weight):
    wp = jnp.pad(weight, ((0, 0), (0, D_PAD - D_MODEL)))
    ids_t = input_ids.T.astype(jnp.int32)
    out = _gather_embed(wp, ids_t)
    return out.transpose(2, 0, 1)


# speedup vs baseline: 1.2701x; 1.0030x over previous
import functools

import jax
import jax.numpy as jnp
from jax import lax
from jax.experimental import pallas as pl
from jax.experimental.pallas import tpu as pltpu
from jax.experimental.pallas import tpu_sc as plsc

D_MODEL = 64
D_PAD = 128
SEQ = 200
BATCH = 4096
TOK_BLK = 256
BLKS = BATCH // TOK_BLK
ITEMS = SEQ * BLKS
NUM_CORES = 2
NUM_SUBCORES = 16
NUM_WORKERS = NUM_CORES * NUM_SUBCORES
ITEMS_PER_W = ITEMS // NUM_WORKERS

def _diag_transpose(src, dst, lane, rot, n_row_groups, n_col_groups):
    @plsc.parallel_loop(0, n_row_groups, unroll=2)
    def trans(rg):
        r0 = rg * 16
        row = lane + r0
        for cg in range(n_col_groups):
            c0 = cg * 16
            for r in range(16):
                col = rot[r] + c0
                vals = plsc.load_gather(src, [row, col])
                plsc.store_scatter(dst, [col, row], vals)


def _gather_embed(wp, ids_t):
    mesh = plsc.VectorSubcoreMesh(core_axis_name="c", subcore_axis_name="s")

    @functools.partial(
        pl.kernel,
        mesh=mesh,
        out_type=jax.ShapeDtypeStruct((SEQ, D_MODEL, BATCH), jnp.float32),
        scratch_types=[
            pltpu.VMEM((2, 2, 128), jnp.int32),
            pltpu.VMEM((2, TOK_BLK, D_PAD), jnp.float32),
            pltpu.VMEM((2, D_MODEL, TOK_BLK), jnp.float32),
            [pltpu.SemaphoreType.DMA] * 4,
        ],
        compiler_params=pltpu.CompilerParams(use_tc_tiling_on_sc=True,
                                             needs_layout_passes=False),
    )
    def k(table_hbm, idx_hbm, out_hbm, idx_v, g_v, t_v, sems):
        sems_g = [sems[0], sems[1]]
        sems_s = [sems[2], sems[3]]
        wid = lax.axis_index("s") * NUM_CORES + lax.axis_index("c")
        item0 = wid * ITEMS_PER_W
        lane = lax.iota(jnp.int32, 16)
        rot = [lax.rem(lane + r, 16) for r in range(16)]

        def pos(i):
            item = item0 + i
            return item // BLKS, (item % BLKS) * TOK_BLK

        def stage(i, b):
            s, b0 = pos(i)
            for j in range(2):
                pltpu.sync_copy(idx_hbm.at[s, pl.ds(b0 + j * 128, 128)],
                                idx_v.at[b, j])
            for j in range(2):
                pltpu.async_copy(table_hbm.at[idx_v.at[b, j]],
                                 g_v.at[b, pl.ds(j * 128, 128)],
                                 sems_g[b])

        def drain(b):
            for j in range(2):
                pltpu.make_async_copy(
                    table_hbm.at[idx_v.at[b, j]],
                    g_v.at[b, pl.ds(j * 128, 128)],
                    sems_g[b]).wait()

        def store(i, b):
            s, b0 = pos(i)
            pltpu.async_copy(t_v.at[b],
                             out_hbm.at[s, :, pl.ds(b0, TOK_BLK)],
                             sems_s[b])

        def wait_store(b):
            pltpu.make_async_copy(t_v.at[b],
                                  out_hbm.at[0, :, pl.ds(0, TOK_BLK)],
                                  sems_s[b]).wait()

        def flush(i, b):
            wait_store(b)
            _diag_transpose(g_v.at[b], t_v.at[b], lane, rot,
                            TOK_BLK // 16, D_MODEL // 16)
            store(i, b)

        stage(0, 0)
        store(0, 0)
        store(1, 1)

        def body2(g, carry):
            i = g * 2
            drain(0)
            stage(i + 1, 1)
            flush(i, 0)
            drain(1)
            stage(i + 2, 0)
            flush(i + 1, 1)
            return carry

        lax.fori_loop(0, (ITEMS_PER_W - 2) // 2, body2, 0)
        i_last = ITEMS_PER_W - 2
        drain(0)
        stage(i_last + 1, 1)
        flush(i_last, 0)
        drain(1)
        flush(i_last + 1, 1)
        wait_store(0)
        wait_store(1)

    return k(wp, ids_t)


def kernel(input_ids, weight):
    wp = jnp.pad(weight, ((0, 0), (0, D_PAD - D_MODEL)))
    ids_t = input_ids.T.astype(jnp.int32)
    out = _gather_embed(wp, ids_t)
    return out.transpose(2, 0, 1)
